# baseline (speedup 1.0000x reference)
import functools

import jax
import jax.numpy as jnp
from jax import lax
from jax.experimental import pallas as pl
from jax.experimental.pallas import tpu as pltpu
from jax.experimental.pallas import tpu_sc as plsc

NC = 2
NS = 16
NW = NC * NS
L = 16
BS = 128


def _sc_body(J, VCH, vxy_hbm, vz_hbm, faces_hbm, out_hbm,
             ia, ib, ic, nx, ny, nz,
             g00, g01, g03, g04, g06, g07,
             g10, g11, g13, g14, g16, g17,
             zbuf, vtxy, vtz, accx, accy, accz, semga, semgb, sems):
    c = lax.axis_index("c")
    s = lax.axis_index("s")
    w = c * NS + s
    VACC = VCH * NS
    base_v = s * VCH
    sl = pl.ds(base_v, VCH)

    lanes = lax.iota(jnp.int32, L)
    zv = jnp.zeros((L,), jnp.float32)

    def _zfill(k, _):
        zbuf[pl.ds(k * L, L)] = zv
        return _

    lax.fori_loop(0, VCH // L, _zfill, None)
    pltpu.sync_copy(zbuf, accx.at[sl])
    pltpu.sync_copy(zbuf, accy.at[sl])
    pltpu.sync_copy(zbuf, accz.at[sl])

    pltpu.sync_copy(vxy_hbm.at[sl], vtxy.at[sl])
    pltpu.sync_copy(vz_hbm.at[sl], vtz.at[sl])

    pltpu.sync_copy(faces_hbm.at[0, w], ia)
    pltpu.sync_copy(faces_hbm.at[1, w], ib)
    pltpu.sync_copy(faces_hbm.at[2, w], ic)

    plsc.subcore_barrier()

    def _fire_gathers(row_a, row_b, row_c, bufs, sem):
        return [
            pltpu.async_copy(vtxy.at[row_a], bufs[0][0], sem),
            pltpu.async_copy(vtz.at[row_a], bufs[0][1], sem),
            pltpu.async_copy(vtxy.at[row_b], bufs[1][0], sem),
            pltpu.async_copy(vtz.at[row_b], bufs[1][1], sem),
            pltpu.async_copy(vtxy.at[row_c], bufs[2][0], sem),
            pltpu.async_copy(vtz.at[row_c], bufs[2][1], sem),
        ]

    himask = jnp.full((L,), -65536, jnp.int32)

    def _unpack(wxy):
        x = plsc.bitcast(wxy & himask, jnp.float32)
        y = plsc.bitcast(lax.shift_left(wxy, 16), jnp.float32)
        return x, y

    def _compute(j, bufs):
        jv = jnp.full((L,), j, jnp.int32)
        for i in range(BS // L):
            col = i * L + lanes
            ii = pl.ds(i * L, L)
            ax, ay = _unpack(bufs[0][0][ii])
            bx, by = _unpack(bufs[1][0][ii])
            cx, cy = _unpack(bufs[2][0][ii])
            az, bz, cz = bufs[0][1][ii], bufs[1][1][ii], bufs[2][1][ii]
            ux, uy, uz = cx - bx, cy - by, cz - bz
            vx, vy, vz = ax - bx, ay - by, az - bz
            plsc.store_scatter(nx, [jv, col], uy * vz - uz * vy)
            plsc.store_scatter(ny, [jv, col], uz * vx - ux * vz)
            plsc.store_scatter(nz, [jv, col], ux * vy - uy * vx)

    def _fire_scatters(j, row_a, row_b, row_c):
        for row in (row_a, row_b, row_c):
            pltpu.async_copy(nx.at[j], accx.at[row], sems, add=True)
            pltpu.async_copy(ny.at[j], accy.at[row], sems, add=True)
            pltpu.async_copy(nz.at[j], accz.at[row], sems, add=True)

    def _wait_scatters(j):
        for row in (ia.at[j], ib.at[j], ic.at[j]):
            pltpu.make_async_copy(nx.at[j], accx.at[row], sems).wait()
            pltpu.make_async_copy(ny.at[j], accy.at[row], sems).wait()
            pltpu.make_async_copy(nz.at[j], accz.at[row], sems).wait()

    bufs0 = ((g00, g01), (g03, g04), (g06, g07))
    bufs1 = ((g10, g11), (g13, g14), (g16, g17))

    def _pair(t, _):
        j0 = 2 * t
        j1 = 2 * t + 1
        ra0, rb0, rc0 = ia.at[j0], ib.at[j0], ic.at[j0]
        ra1, rb1, rc1 = ia.at[j1], ib.at[j1], ic.at[j1]
        da = _fire_gathers(ra0, rb0, rc0, bufs0, semga)
        db = _fire_gathers(ra1, rb1, rc1, bufs1, semgb)
        for d in da:
            d.wait()
        _compute(j0, bufs0)

        @pl.when(t >= 1)
        def _drain_prev():
            _wait_scatters(2 * t - 2)
            _wait_scatters(2 * t - 1)

        _fire_scatters(j0, ra0, rb0, rc0)
        for d in db:
            d.wait()
        _compute(j1, bufs1)
        _fire_scatters(j1, ra1, rb1, rc1)
        return _

    lax.fori_loop(0, J // 2, _pair, None)
    _wait_scatters(J - 2)
    _wait_scatters(J - 1)

    plsc.subcore_barrier()

    pltpu.sync_copy(accx.at[sl], out_hbm.at[pl.ds((c * 3 + 0) * VACC + base_v, VCH)])
    pltpu.sync_copy(accy.at[sl], out_hbm.at[pl.ds((c * 3 + 1) * VACC + base_v, VCH)])
    pltpu.sync_copy(accz.at[sl], out_hbm.at[pl.ds((c * 3 + 2) * VACC + base_v, VCH)])


def _tc_norm(V, q_ref, o_ref):
    p = q_ref[0] + q_ref[1]
    ss = jnp.sum(p * p, axis=0, keepdims=True)
    scale = 1.0 / jnp.maximum(jnp.sqrt(ss), 1e-6)
    o_ref[...] = (p * scale)[:, :V]


def kernel(verts, faces):
    V = verts.shape[0]
    F = faces.shape[0]
    J = -(-F // (NW * BS))
    J = -(-J // 4) * 4
    Fp = NW * J * BS
    VCH = (-(-V // (NS * BS))) * BS
    if VCH * NS == V:
        VCH += BS
    VACC = VCH * NS

    faces = faces.astype(jnp.int32)
    pad_idx = V + (jnp.arange(Fp - F, dtype=jnp.int32) % (VACC - V))
    f_pad = jnp.concatenate(
        [faces, jnp.broadcast_to(pad_idx[:, None], (Fp - F, 3))], axis=0)
    f_soa = f_pad.T.reshape(3, NW, J, BS)
    v_soa = jnp.zeros((3, VACC), jnp.float32).at[:, :V].set(verts.T)
    xb = lax.bitcast_convert_type(
        v_soa[0].astype(jnp.bfloat16), jnp.uint16).astype(jnp.uint32)
    yb = lax.bitcast_convert_type(
        v_soa[1].astype(jnp.bfloat16), jnp.uint16).astype(jnp.uint32)
    vxy = lax.bitcast_convert_type((xb << 16) | yb, jnp.int32)
    vz = v_soa[2]

    mesh = plsc.VectorSubcoreMesh(
        core_axis_name="c", subcore_axis_name="s",
        num_cores=NC, num_subcores=NS)
    sc = pl.kernel(
        functools.partial(_sc_body, J, VCH),
        out_type=jax.ShapeDtypeStruct((NC * 3 * VACC,), jnp.float32),
        mesh=mesh,
        compiler_params=pltpu.CompilerParams(needs_layout_passes=False),
        scratch_types=[
            pltpu.VMEM((J, BS), jnp.int32),
            pltpu.VMEM((J, BS), jnp.int32),
            pltpu.VMEM((J, BS), jnp.int32),
            pltpu.VMEM((J, BS), jnp.float32),
            pltpu.VMEM((J, BS), jnp.float32),
            pltpu.VMEM((J, BS), jnp.float32),
        ] + [pltpu.VMEM((BS,), jnp.int32),
             pltpu.VMEM((BS,), jnp.float32)] * 6
        + [
            pltpu.VMEM((VCH,), jnp.float32),
            pltpu.VMEM_SHARED((VACC,), jnp.int32),
            pltpu.VMEM_SHARED((VACC,), jnp.float32),
            pltpu.VMEM_SHARED((VACC,), jnp.float32),
            pltpu.VMEM_SHARED((VACC,), jnp.float32),
            pltpu.VMEM_SHARED((VACC,), jnp.float32),
            pltpu.SemaphoreType.DMA,
            pltpu.SemaphoreType.DMA,
            pltpu.SemaphoreType.DMA,
        ],
    )
    partials = sc(vxy, vz, f_soa).reshape(NC, 3, VACC)

    out = pl.pallas_call(
        functools.partial(_tc_norm, V),
        out_shape=jax.ShapeDtypeStruct((3, V), jnp.float32),
    )(partials)
    return out.T


# speedup vs baseline: 1.1204x; 1.1204x over previous
import functools

import jax
import jax.numpy as jnp
from jax import lax
from jax.experimental import pallas as pl
from jax.experimental.pallas import tpu as pltpu
from jax.experimental.pallas import tpu_sc as plsc

NC = 2
NS = 16
NW = NC * NS
L = 16
BS = 128


def _sc_body(J, VCH, verts_hbm, faces_hbm, out_hbm,
             ia, ib, ic, nx, ny, nz,
             g00, g01, g02, g03, g04, g05, g06, g07, g08,
             g10, g11, g12, g13, g14, g15, g16, g17, g18,
             zbuf, vtx, vty, vtz, accx, accy, accz, semga, semgb, sems):
    c = lax.axis_index("c")
    s = lax.axis_index("s")
    w = c * NS + s
    VACC = VCH * NS
    base_v = s * VCH
    sl = pl.ds(base_v, VCH)

    lanes = lax.iota(jnp.int32, L)
    zv = jnp.zeros((L,), jnp.float32)

    def _zfill(k, _):
        zbuf[pl.ds(k * L, L)] = zv
        return _

    lax.fori_loop(0, VCH // L, _zfill, None)
    pltpu.sync_copy(zbuf, accx.at[sl])
    pltpu.sync_copy(zbuf, accy.at[sl])
    pltpu.sync_copy(zbuf, accz.at[sl])

    pltpu.sync_copy(verts_hbm.at[pl.ds(0 * VACC + base_v, VCH)], vtx.at[sl])
    pltpu.sync_copy(verts_hbm.at[pl.ds(1 * VACC + base_v, VCH)], vty.at[sl])
    pltpu.sync_copy(verts_hbm.at[pl.ds(2 * VACC + base_v, VCH)], vtz.at[sl])

    pltpu.sync_copy(faces_hbm.at[0, w], ia)
    pltpu.sync_copy(faces_hbm.at[1, w], ib)
    pltpu.sync_copy(faces_hbm.at[2, w], ic)

    plsc.subcore_barrier()

    def _fire_gathers(row_a, row_b, row_c, bufs, sem):
        return [
            pltpu.async_copy(vtx.at[row_a], bufs[0][0], sem),
            pltpu.async_copy(vty.at[row_a], bufs[0][1], sem),
            pltpu.async_copy(vtz.at[row_a], bufs[0][2], sem),
            pltpu.async_copy(vtx.at[row_b], bufs[1][0], sem),
            pltpu.async_copy(vty.at[row_b], bufs[1][1], sem),
            pltpu.async_copy(vtz.at[row_b], bufs[1][2], sem),
            pltpu.async_copy(vtx.at[row_c], bufs[2][0], sem),
            pltpu.async_copy(vty.at[row_c], bufs[2][1], sem),
            pltpu.async_copy(vtz.at[row_c], bufs[2][2], sem),
        ]

    def _compute(j, bufs):
        jv = jnp.full((L,), j, jnp.int32)
        for i in range(BS // L):
            col = i * L + lanes
            ii = pl.ds(i * L, L)
            ax, ay, az = bufs[0][0][ii], bufs[0][1][ii], bufs[0][2][ii]
            bx, by, bz = bufs[1][0][ii], bufs[1][1][ii], bufs[1][2][ii]
            cx, cy, cz = bufs[2][0][ii], bufs[2][1][ii], bufs[2][2][ii]
            ux, uy, uz = cx - bx, cy - by, cz - bz
            vx, vy, vz = ax - bx, ay - by, az - bz
            plsc.store_scatter(nx, [jv, col], uy * vz - uz * vy)
            plsc.store_scatter(ny, [jv, col], uz * vx - ux * vz)
            plsc.store_scatter(nz, [jv, col], ux * vy - uy * vx)

    def _fire_scatters(j, row_a, row_b, row_c):
        for row in (row_a, row_b, row_c):
            pltpu.async_copy(nx.at[j], accx.at[row], sems, add=True)
            pltpu.async_copy(ny.at[j], accy.at[row], sems, add=True)
            pltpu.async_copy(nz.at[j], accz.at[row], sems, add=True)

    def _wait_scatters(j):
        for row in (ia.at[j], ib.at[j], ic.at[j]):
            pltpu.make_async_copy(nx.at[j], accx.at[row], sems).wait()
            pltpu.make_async_copy(ny.at[j], accy.at[row], sems).wait()
            pltpu.make_async_copy(nz.at[j], accz.at[row], sems).wait()

    bufs0 = ((g00, g01, g02), (g03, g04, g05), (g06, g07, g08))
    bufs1 = ((g10, g11, g12), (g13, g14, g15), (g16, g17, g18))

    def _wait_gathers(j, bufs, sem):
        rows = (ia.at[j], ib.at[j], ic.at[j])
        for corner in range(3):
            pltpu.make_async_copy(vtx.at[rows[corner]], bufs[corner][0], sem).wait()
            pltpu.make_async_copy(vty.at[rows[corner]], bufs[corner][1], sem).wait()
            pltpu.make_async_copy(vtz.at[rows[corner]], bufs[corner][2], sem).wait()

    _fire_gathers(ia.at[0], ib.at[0], ic.at[0], bufs0, semga)

    def _pair(t, _):
        j0 = 2 * t
        j1 = 2 * t + 1
        ra0, rb0, rc0 = ia.at[j0], ib.at[j0], ic.at[j0]
        ra1, rb1, rc1 = ia.at[j1], ib.at[j1], ic.at[j1]
        _fire_gathers(ra1, rb1, rc1, bufs1, semgb)
        _wait_gathers(j0, bufs0, semga)
        _compute(j0, bufs0)

        @pl.when(t + 1 < J // 2)
        def _prefetch_next():
            _fire_gathers(ia.at[j0 + 2], ib.at[j0 + 2], ic.at[j0 + 2],
                          bufs0, semga)

        _fire_scatters(j0, ra0, rb0, rc0)
        _wait_gathers(j1, bufs1, semgb)
        _compute(j1, bufs1)
        _fire_scatters(j1, ra1, rb1, rc1)

        @pl.when(t >= 1)
        def _drain_prev():
            _wait_scatters(2 * t - 2)
            _wait_scatters(2 * t - 1)

        return _

    lax.fori_loop(0, J // 2, _pair, None)
    _wait_scatters(J - 2)
    _wait_scatters(J - 1)

    plsc.subcore_barrier()

    pltpu.sync_copy(accx.at[sl], out_hbm.at[pl.ds((c * 3 + 0) * VACC + base_v, VCH)])
    pltpu.sync_copy(accy.at[sl], out_hbm.at[pl.ds((c * 3 + 1) * VACC + base_v, VCH)])
    pltpu.sync_copy(accz.at[sl], out_hbm.at[pl.ds((c * 3 + 2) * VACC + base_v, VCH)])


def _tc_norm(V, q_ref, o_ref):
    p = q_ref[0] + q_ref[1]
    ss = jnp.sum(p * p, axis=0, keepdims=True)
    scale = 1.0 / jnp.maximum(jnp.sqrt(ss), 1e-6)
    o_ref[...] = (p * scale)[:, :V]


def kernel(verts, faces):
    V = verts.shape[0]
    F = faces.shape[0]
    J = -(-F // (NW * BS))
    J = -(-J // 4) * 4
    Fp = NW * J * BS
    VCH = (-(-V // (NS * BS))) * BS
    if VCH * NS == V:
        VCH += BS
    VACC = VCH * NS

    faces = faces.astype(jnp.int32)
    pad_idx = V + (jnp.arange(Fp - F, dtype=jnp.int32) % (VACC - V))
    f_pad = jnp.concatenate(
        [faces, jnp.broadcast_to(pad_idx[:, None], (Fp - F, 3))], axis=0)
    f_soa = f_pad.T.reshape(3, NW, J, BS)
    v_soa = jnp.zeros((3, VACC), jnp.float32).at[:, :V].set(verts.T)

    mesh = plsc.VectorSubcoreMesh(
        core_axis_name="c", subcore_axis_name="s",
        num_cores=NC, num_subcores=NS)
    sc = pl.kernel(
        functools.partial(_sc_body, J, VCH),
        out_type=jax.ShapeDtypeStruct((NC * 3 * VACC,), jnp.float32),
        mesh=mesh,
        compiler_params=pltpu.CompilerParams(needs_layout_passes=False),
        scratch_types=[
            pltpu.VMEM((J, BS), jnp.int32),
            pltpu.VMEM((J, BS), jnp.int32),
            pltpu.VMEM((J, BS), jnp.int32),
            pltpu.VMEM((J, BS), jnp.float32),
            pltpu.VMEM((J, BS), jnp.float32),
            pltpu.VMEM((J, BS), jnp.float32),
        ] + [pltpu.VMEM((BS,), jnp.float32)] * 18
        + [
            pltpu.VMEM((VCH,), jnp.float32),
            pltpu.VMEM_SHARED((VACC,), jnp.float32),
            pltpu.VMEM_SHARED((VACC,), jnp.float32),
            pltpu.VMEM_SHARED((VACC,), jnp.float32),
            pltpu.VMEM_SHARED((VACC,), jnp.float32),
            pltpu.VMEM_SHARED((VACC,), jnp.float32),
            pltpu.VMEM_SHARED((VACC,), jnp.float32),
            pltpu.SemaphoreType.DMA,
            pltpu.SemaphoreType.DMA,
            pltpu.SemaphoreType.DMA,
        ],
    )
    partials = sc(v_soa.reshape(-1), f_soa).reshape(NC, 3, VACC)

    out = pl.pallas_call(
        functools.partial(_tc_norm, V),
        out_shape=jax.ShapeDtypeStruct((3, V), jnp.float32),
    )(partials)
    return out.T
